# baseline (speedup 1.0000x reference)
import functools

import jax
import jax.numpy as jnp
from jax import lax
from jax.experimental import pallas as pl
from jax.experimental.pallas import tpu as pltpu
from jax.experimental.pallas import tpu_sc as plsc

N_USERS = 6000
N_ITEMS = 4000
N = N_USERS + N_ITEMS
E = 320000
D = 128

NC = 2
NS = 16
NW = NC * NS

CH = 128
BCH = 16
K_PUSH = 80
EW_PAD = K_PUSH * CH
K_DEG = -(-(E // NS) // CH)
ET_PAD = K_DEG * CH

N_ACC = 10240
ROWS_T = N_ACC // NS
DUMP = N

_mesh = plsc.VectorSubcoreMesh(core_axis_name="c", subcore_axis_name="s")


def _zero_fill(zbuf, rows, width):
    zero = jnp.zeros((16,), jnp.float32)
    for i in range(rows):
        for l in range(width // 16):
            zbuf[i, pl.ds(l * 16, 16)] = zero


@functools.partial(
    pl.kernel,
    out_type=jax.ShapeDtypeStruct((NC, N_ACC, D), jnp.float32),
    mesh=_mesh,
    scratch_types=[
        pltpu.VMEM((K_DEG, CH), jnp.int32),
        pltpu.VMEM((CH, D), jnp.float32),
        pltpu.VMEM((64, D), jnp.float32),
        pltpu.VMEM_SHARED((N_ACC, D), jnp.float32),
    ],
)
def _deg_kernel(idx_hbm, out_hbm, idx_v, ones_v, zbuf, acc_sh):
    c = lax.axis_index("c")
    s = lax.axis_index("s")
    one = jnp.full((16,), 1.0, jnp.float32)
    for i in range(CH):
        for l in range(D // 16):
            ones_v[i, pl.ds(l * 16, 16)] = one
    _zero_fill(zbuf, 64, D)
    for t in range(ROWS_T // 64):
        pltpu.sync_copy(zbuf, acc_sh.at[pl.ds(s * ROWS_T + t * 64, 64)])
    pltpu.sync_copy(idx_hbm.at[c * NS + s], idx_v)
    plsc.subcore_barrier()

    def step(j, carry):
        pltpu.sync_copy(ones_v, acc_sh.at[idx_v.at[j]], add=True)
        return carry

    lax.fori_loop(0, K_DEG, step, 0)
    plsc.subcore_barrier()
    pltpu.sync_copy(acc_sh.at[pl.ds(s * ROWS_T, ROWS_T)],
                    out_hbm.at[c, pl.ds(s * ROWS_T, ROWS_T)])


@functools.partial(
    pl.kernel,
    out_type=jax.ShapeDtypeStruct((NC, N_ACC, D), jnp.float32),
    mesh=_mesh,
    scratch_types=[
        pltpu.VMEM((BCH, CH), jnp.int32),
        pltpu.VMEM((BCH, CH), jnp.int32),
        pltpu.VMEM((BCH, CH), jnp.int32),
        pltpu.VMEM((BCH, CH), jnp.int32),
        pltpu.VMEM((CH, D), jnp.float32),
        pltpu.VMEM((CH, D), jnp.float32),
        pltpu.VMEM_SHARED((N_ACC, D), jnp.float32),
        pltpu.SemaphoreType.DMA,
        pltpu.SemaphoreType.DMA,
        pltpu.SemaphoreType.DMA,
        pltpu.SemaphoreType.DMA,
    ],
)
def _push_kernel(src_hbm, dst_hbm, table_hbm, out_hbm,
                 src_a, src_b, dst_a, dst_b, row_a, row_b, acc_sh,
                 sem_ra, sem_rb, sem_ia, sem_ib):
    c = lax.axis_index("c")
    s = lax.axis_index("s")
    wid = c * NS + s
    _zero_fill(row_a, CH, D)
    for t in range(ROWS_T // CH):
        pltpu.sync_copy(row_a, acc_sh.at[pl.ds(s * ROWS_T + t * CH, CH)])
    pltpu.sync_copy(src_hbm.at[wid, pl.ds(0, BCH)], src_a)
    pltpu.sync_copy(dst_hbm.at[wid, pl.ds(0, BCH)], dst_a)
    pltpu.async_copy(src_hbm.at[wid, pl.ds(BCH, BCH)], src_b, sem_ib)
    pltpu.async_copy(dst_hbm.at[wid, pl.ds(BCH, BCH)], dst_b, sem_ib)
    plsc.subcore_barrier()

    sbufs = (src_a, src_b)
    dbufs = (dst_a, dst_b)
    rbufs = (row_a, row_b)
    rsems = (sem_ra, sem_rb)
    pltpu.async_copy(table_hbm.at[src_a.at[0]], row_a, sem_ra)
    for j in range(K_PUSH):
        blk, off = divmod(j, BCH)
        rb, rs = rbufs[j % 2], rsems[j % 2]
        db = dbufs[blk % 2]
        pltpu.make_async_copy(table_hbm.at[src_a.at[0]], rb, rs).wait()
        if j + 1 < K_PUSH:
            nblk, noff = divmod(j + 1, BCH)
            nsb = sbufs[nblk % 2]
            if nblk != blk:
                nis = sem_ib if nblk % 2 else sem_ia
                pltpu.make_async_copy(src_hbm.at[wid, pl.ds(0, BCH)],
                                      nsb, nis).wait()
                pltpu.make_async_copy(dst_hbm.at[wid, pl.ds(0, BCH)],
                                      dbufs[nblk % 2], nis).wait()
            pltpu.async_copy(table_hbm.at[nsb.at[noff]],
                             rbufs[(j + 1) % 2], rsems[(j + 1) % 2])
        pltpu.sync_copy(rb, acc_sh.at[db.at[off]], add=True)
        if off == BCH - 1 and (blk + 2) * BCH < K_PUSH:
            pis = sem_ib if blk % 2 else sem_ia
            pltpu.async_copy(
                src_hbm.at[wid, pl.ds((blk + 2) * BCH, BCH)], sbufs[blk % 2], pis)
            pltpu.async_copy(
                dst_hbm.at[wid, pl.ds((blk + 2) * BCH, BCH)], dbufs[blk % 2], pis)
    plsc.subcore_barrier()
    pltpu.sync_copy(acc_sh.at[pl.ds(s * ROWS_T, ROWS_T)],
                    out_hbm.at[c, pl.ds(s * ROWS_T, ROWS_T)])


def _prep_body(all_ref, dego_ref, out_ref):
    rs = lax.rsqrt(jnp.maximum(dego_ref[...], 1.0))
    out_ref[...] = all_ref[...] * rs


BR = 1000


def _prep_call(all0, deg_out):
    return pl.pallas_call(
        _prep_body,
        out_shape=jax.ShapeDtypeStruct((N, D), jnp.float32),
        grid=(N // BR,),
        in_specs=[
            pl.BlockSpec((BR, D), lambda i: (i, 0)),
            pl.BlockSpec((BR, 1), lambda i: (i, 0)),
        ],
        out_specs=pl.BlockSpec((BR, D), lambda i: (i, 0)),
    )(all0, deg_out)


def _layer_body(acc_ref, all_ref, degi_ref, dego_ref,
                w1_ref, b1_ref, w2_ref, b2_ref,
                raw_ref, norm_ref, scaled_ref):
    nei = (acc_ref[0] + acc_ref[1]) * lax.rsqrt(jnp.maximum(degi_ref[...], 1.0))
    alle = all_ref[...]
    w1b = w1_ref[...] + b1_ref[...]
    w2b = w2_ref[...] + b2_ref[...]
    sum_e = jnp.dot(nei + alle, w1b, preferred_element_type=jnp.float32)
    sum_e = jnp.where(sum_e >= 0, sum_e, 0.2 * sum_e)
    bi = jnp.dot(nei * alle, w2b, preferred_element_type=jnp.float32)
    bi = jnp.where(bi >= 0, bi, 0.2 * bi)
    new = sum_e + bi
    nrm = jnp.sqrt(jnp.sum(new * new, axis=1, keepdims=True))
    raw_ref[...] = new
    norm_ref[...] = new / jnp.maximum(nrm, 1e-12)
    scaled_ref[...] = new * lax.rsqrt(jnp.maximum(dego_ref[...], 1.0))


def _layer_call(acc, alle, deg_in, deg_out, W1, b1, W2, b2):
    return pl.pallas_call(
        _layer_body,
        out_shape=(
            jax.ShapeDtypeStruct((N, D), jnp.float32),
            jax.ShapeDtypeStruct((N, D), jnp.float32),
            jax.ShapeDtypeStruct((N, D), jnp.float32),
        ),
        grid=(N // BR,),
        in_specs=[
            pl.BlockSpec((NC, BR, D), lambda i: (0, i, 0)),
            pl.BlockSpec((BR, D), lambda i: (i, 0)),
            pl.BlockSpec((BR, 1), lambda i: (i, 0)),
            pl.BlockSpec((BR, 1), lambda i: (i, 0)),
            pl.BlockSpec((D, D), lambda i: (0, 0)),
            pl.BlockSpec((1, D), lambda i: (0, 0)),
            pl.BlockSpec((D, D), lambda i: (0, 0)),
            pl.BlockSpec((1, D), lambda i: (0, 0)),
        ],
        out_specs=(
            pl.BlockSpec((BR, D), lambda i: (i, 0)),
            pl.BlockSpec((BR, D), lambda i: (i, 0)),
            pl.BlockSpec((BR, D), lambda i: (i, 0)),
        ),
    )(acc, alle, deg_in, deg_out, W1, b1, W2, b2)


def kernel(edge_index, embed_user, embed_item,
           W1_0, b1_0, W2_0, b2_0, W1_1, b1_1, W2_1, b2_1):
    src = edge_index[0].astype(jnp.int32)
    dst = edge_index[1].astype(jnp.int32)
    all0 = jnp.concatenate([embed_user, embed_item], axis=0)

    pad_d = jnp.full((NS * ET_PAD - E,), DUMP, jnp.int32)
    src_d = jnp.concatenate([src, pad_d]).reshape(NS, K_DEG, CH)
    dst_d = jnp.concatenate([dst, pad_d]).reshape(NS, K_DEG, CH)
    idx_deg = jnp.concatenate([src_d, dst_d], axis=0)
    deg = _deg_kernel(idx_deg)
    deg_out = deg[0, :N, :1]
    deg_in = deg[1, :N, :1]

    scaled0 = _prep_call(all0, deg_out)

    pad_s = jnp.zeros((NW * EW_PAD - E,), jnp.int32)
    pad_t = jnp.full((NW * EW_PAD - E,), DUMP, jnp.int32)
    srcp = jnp.concatenate([src, pad_s]).reshape(NW, K_PUSH, CH)
    dstp = jnp.concatenate([dst, pad_t]).reshape(NW, K_PUSH, CH)

    acc1 = _push_kernel(srcp, dstp, scaled0)[:, :N]
    raw1, norm1, scaled1 = _layer_call(acc1, all0, deg_in, deg_out,
                                       W1_0, b1_0, W2_0, b2_0)
    acc2 = _push_kernel(srcp, dstp, scaled1)[:, :N]
    _, norm2, _ = _layer_call(acc2, raw1, deg_in, deg_out,
                              W1_1, b1_1, W2_1, b2_1)

    final = jnp.concatenate([all0, norm1, norm2], axis=1)
    return (final[:N_USERS], final[N_USERS:])


# speedup vs baseline: 1.0618x; 1.0618x over previous
import functools

import jax
import jax.numpy as jnp
from jax import lax
from jax.experimental import pallas as pl
from jax.experimental.pallas import tpu as pltpu
from jax.experimental.pallas import tpu_sc as plsc

N_USERS = 6000
N_ITEMS = 4000
N = N_USERS + N_ITEMS
E = 320000
D = 128

NC = 2
NS = 16
NW = NC * NS

CH = 128
BCH = 16
KA = 128
KB = 32
TOT_CH = NS * (KA + KB)
K_DEG = -(-(E // NS) // CH)
ET_PAD = K_DEG * CH

N_ACC = 10240
ROWS_T = N_ACC // NS
DUMP = N

_mesh = plsc.VectorSubcoreMesh(core_axis_name="c", subcore_axis_name="s")


def _zero_fill(zbuf, rows, width):
    zero = jnp.zeros((16,), jnp.float32)
    for i in range(rows):
        for l in range(width // 16):
            zbuf[i, pl.ds(l * 16, 16)] = zero


@functools.partial(
    pl.kernel,
    out_type=jax.ShapeDtypeStruct((NC, N_ACC, D), jnp.float32),
    mesh=_mesh,
    scratch_types=[
        pltpu.VMEM((K_DEG, CH), jnp.int32),
        pltpu.VMEM((CH, D), jnp.float32),
        pltpu.VMEM((64, D), jnp.float32),
        pltpu.VMEM_SHARED((N_ACC, D), jnp.float32),
    ],
)
def _deg_kernel(idx_hbm, out_hbm, idx_v, ones_v, zbuf, acc_sh):
    c = lax.axis_index("c")
    s = lax.axis_index("s")
    one = jnp.full((16,), 1.0, jnp.float32)
    for i in range(CH):
        for l in range(D // 16):
            ones_v[i, pl.ds(l * 16, 16)] = one
    _zero_fill(zbuf, 64, D)
    for t in range(ROWS_T // 64):
        pltpu.sync_copy(zbuf, acc_sh.at[pl.ds(s * ROWS_T + t * 64, 64)])
    pltpu.sync_copy(idx_hbm.at[c * NS + s], idx_v)
    plsc.subcore_barrier()

    def step(j, carry):
        pltpu.sync_copy(ones_v, acc_sh.at[idx_v.at[j]], add=True)
        return carry

    lax.fori_loop(0, K_DEG, step, 0)
    plsc.subcore_barrier()
    pltpu.sync_copy(acc_sh.at[pl.ds(s * ROWS_T, ROWS_T)],
                    out_hbm.at[c, pl.ds(s * ROWS_T, ROWS_T)])


@functools.partial(
    pl.kernel,
    out_type=jax.ShapeDtypeStruct((NC, N_ACC, D), jnp.float32),
    mesh=_mesh,
    scratch_types=[
        pltpu.VMEM((BCH, CH), jnp.int32),
        pltpu.VMEM((BCH, CH), jnp.int32),
        pltpu.VMEM((BCH, CH), jnp.int32),
        pltpu.VMEM((BCH, CH), jnp.int32),
        pltpu.VMEM((CH, D), jnp.float32),
        pltpu.VMEM((CH, D), jnp.float32),
        pltpu.VMEM_SHARED((N_ACC, D), jnp.float32),
        pltpu.SemaphoreType.DMA,
        pltpu.SemaphoreType.DMA,
        pltpu.SemaphoreType.DMA,
        pltpu.SemaphoreType.DMA,
    ],
)
def _push_kernel(src_hbm, dst_hbm, table_hbm, out_hbm,
                 src_a, src_b, dst_a, dst_b, row_a, row_b, acc_sh,
                 sem_ra, sem_rb, sem_ia, sem_ib):
    c = lax.axis_index("c")
    s = lax.axis_index("s")
    start = jnp.where(c == 0, s * KA, NS * KA + s * KB)
    npairs = jnp.where(c == 0, KA // (2 * BCH), KB // (2 * BCH))
    _zero_fill(row_a, CH, D)
    for t in range(ROWS_T // CH):
        pltpu.sync_copy(row_a, acc_sh.at[pl.ds(s * ROWS_T + t * CH, CH)])
    pltpu.async_copy(src_hbm.at[pl.ds(start, BCH)], src_a, sem_ia)
    pltpu.async_copy(dst_hbm.at[pl.ds(start, BCH)], dst_a, sem_ia)
    plsc.subcore_barrier()

    def _drain_idx(sbuf, dbuf, sem):
        pltpu.make_async_copy(src_hbm.at[pl.ds(0, BCH)], sbuf, sem).wait()
        pltpu.make_async_copy(dst_hbm.at[pl.ds(0, BCH)], dbuf, sem).wait()

    def _block(sbuf, dbuf, nsbuf, ndbuf, nsem, nbase):
        pltpu.async_copy(src_hbm.at[pl.ds(nbase, BCH)], nsbuf, nsem)
        pltpu.async_copy(dst_hbm.at[pl.ds(nbase, BCH)], ndbuf, nsem)
        rbufs = (row_a, row_b)
        rsems = (sem_ra, sem_rb)
        pltpu.async_copy(table_hbm.at[sbuf.at[0]], row_a, sem_ra)
        for off in range(BCH):
            rb, rs = rbufs[off % 2], rsems[off % 2]
            pltpu.make_async_copy(table_hbm.at[sbuf.at[0]], rb, rs).wait()
            if off + 1 < BCH:
                pltpu.async_copy(table_hbm.at[sbuf.at[off + 1]],
                                 rbufs[(off + 1) % 2], rsems[(off + 1) % 2])
            pltpu.sync_copy(rb, acc_sh.at[dbuf.at[off]], add=True)

    def pair(p, carry):
        base = start + p * 2 * BCH
        _drain_idx(src_a, dst_a, sem_ia)
        _block(src_a, dst_a, src_b, dst_b, sem_ib, base + BCH)
        _drain_idx(src_b, dst_b, sem_ib)
        nxt = jnp.minimum(base + 2 * BCH, start + npairs * 2 * BCH - BCH)
        _block(src_b, dst_b, src_a, dst_a, sem_ia, nxt)
        return carry

    lax.fori_loop(0, npairs, pair, 0)
    _drain_idx(src_a, dst_a, sem_ia)
    plsc.subcore_barrier()
    pltpu.sync_copy(acc_sh.at[pl.ds(s * ROWS_T, ROWS_T)],
                    out_hbm.at[c, pl.ds(s * ROWS_T, ROWS_T)])


def _prep_body(all_ref, dego_ref, out_ref):
    rs = lax.rsqrt(jnp.maximum(dego_ref[...], 1.0))
    out_ref[...] = all_ref[...] * rs


BR = 1000


def _prep_call(all0, deg_out):
    return pl.pallas_call(
        _prep_body,
        out_shape=jax.ShapeDtypeStruct((N, D), jnp.float32),
        grid=(N // BR,),
        in_specs=[
            pl.BlockSpec((BR, D), lambda i: (i, 0)),
            pl.BlockSpec((BR, 1), lambda i: (i, 0)),
        ],
        out_specs=pl.BlockSpec((BR, D), lambda i: (i, 0)),
    )(all0, deg_out)


def _layer_body(acc_ref, all_ref, degi_ref, dego_ref,
                w1_ref, b1_ref, w2_ref, b2_ref,
                raw_ref, norm_ref, scaled_ref):
    nei = (acc_ref[0] + acc_ref[1]) * lax.rsqrt(jnp.maximum(degi_ref[...], 1.0))
    alle = all_ref[...]
    w1b = w1_ref[...] + b1_ref[...]
    w2b = w2_ref[...] + b2_ref[...]
    sum_e = jnp.dot(nei + alle, w1b, preferred_element_type=jnp.float32)
    sum_e = jnp.where(sum_e >= 0, sum_e, 0.2 * sum_e)
    bi = jnp.dot(nei * alle, w2b, preferred_element_type=jnp.float32)
    bi = jnp.where(bi >= 0, bi, 0.2 * bi)
    new = sum_e + bi
    nrm = jnp.sqrt(jnp.sum(new * new, axis=1, keepdims=True))
    raw_ref[...] = new
    norm_ref[...] = new / jnp.maximum(nrm, 1e-12)
    scaled_ref[...] = new * lax.rsqrt(jnp.maximum(dego_ref[...], 1.0))


def _layer_call(acc, alle, deg_in, deg_out, W1, b1, W2, b2):
    return pl.pallas_call(
        _layer_body,
        out_shape=(
            jax.ShapeDtypeStruct((N, D), jnp.float32),
            jax.ShapeDtypeStruct((N, D), jnp.float32),
            jax.ShapeDtypeStruct((N, D), jnp.float32),
        ),
        grid=(N // BR,),
        in_specs=[
            pl.BlockSpec((NC, BR, D), lambda i: (0, i, 0)),
            pl.BlockSpec((BR, D), lambda i: (i, 0)),
            pl.BlockSpec((BR, 1), lambda i: (i, 0)),
            pl.BlockSpec((BR, 1), lambda i: (i, 0)),
            pl.BlockSpec((D, D), lambda i: (0, 0)),
            pl.BlockSpec((1, D), lambda i: (0, 0)),
            pl.BlockSpec((D, D), lambda i: (0, 0)),
            pl.BlockSpec((1, D), lambda i: (0, 0)),
        ],
        out_specs=(
            pl.BlockSpec((BR, D), lambda i: (i, 0)),
            pl.BlockSpec((BR, D), lambda i: (i, 0)),
            pl.BlockSpec((BR, D), lambda i: (i, 0)),
        ),
    )(acc, alle, deg_in, deg_out, W1, b1, W2, b2)


def kernel(edge_index, embed_user, embed_item,
           W1_0, b1_0, W2_0, b2_0, W1_1, b1_1, W2_1, b2_1):
    src = edge_index[0].astype(jnp.int32)
    dst = edge_index[1].astype(jnp.int32)
    all0 = jnp.concatenate([embed_user, embed_item], axis=0)

    pad_d = jnp.full((NS * ET_PAD - E,), DUMP, jnp.int32)
    src_d = jnp.concatenate([src, pad_d]).reshape(NS, K_DEG, CH)
    dst_d = jnp.concatenate([dst, pad_d]).reshape(NS, K_DEG, CH)
    idx_deg = jnp.concatenate([src_d, dst_d], axis=0)
    deg = _deg_kernel(idx_deg)
    deg_out = deg[0, :N, :1]
    deg_in = deg[1, :N, :1]

    scaled0 = _prep_call(all0, deg_out)

    pad_s = jnp.zeros((TOT_CH * CH - E,), jnp.int32)
    pad_t = jnp.full((TOT_CH * CH - E,), DUMP, jnp.int32)
    srcp = jnp.concatenate([src, pad_s]).reshape(TOT_CH, CH)
    dstp = jnp.concatenate([dst, pad_t]).reshape(TOT_CH, CH)

    acc1 = _push_kernel(srcp, dstp, scaled0)[:, :N]
    raw1, norm1, scaled1 = _layer_call(acc1, all0, deg_in, deg_out,
                                       W1_0, b1_0, W2_0, b2_0)
    acc2 = _push_kernel(srcp, dstp, scaled1)[:, :N]
    _, norm2, _ = _layer_call(acc2, raw1, deg_in, deg_out,
                              W1_1, b1_1, W2_1, b2_1)

    final = jnp.concatenate([all0, norm1, norm2], axis=1)
    return (final[:N_USERS], final[N_USERS:])
